# baseline (speedup 1.0000x reference)
import functools

import jax
import jax.numpy as jnp
from jax import lax
from jax.experimental import pallas as pl
from jax.experimental.pallas import tpu as pltpu
from jax.experimental.pallas import tpu_sc as plsc

_NC = 2
_NS = 16
_L = 16


def _mesh():
    return plsc.VectorSubcoreMesh(
        core_axis_name="c", subcore_axis_name="s",
        num_cores=_NC, num_subcores=_NS)


def _zero_vmem(ref, n):
    def body(i, carry):
        off = pl.multiple_of(i * _L, _L)
        ref[pl.ds(off, _L)] = jnp.zeros((_L,), jnp.float32)
        return carry
    lax.fori_loop(0, n // _L, body, 0)


def _build_a_body(E, N, CH, U, ea_hbm, src_hbm, a_hbm,
                  acc, segbuf, vsrc, vval, vred, shared, sem):
    c = lax.axis_index("c")
    s = lax.axis_index("s")
    col_local = s // 5
    slot = s % 5
    col = jnp.where(c == 0, col_local, 3 + col_local)
    active = jnp.where(c == 0, s < 15, s < 10)
    SLOT_E = E // 5
    NCH = SLOT_E // CH
    GRP_IT = CH // (_L * U)
    lo = slot * SLOT_E

    @pl.when(active)
    def _():
        _zero_vmem(acc, N)

        def issue(ci, half):
            off = pl.multiple_of(lo + ci * CH, _L)
            boff = pl.multiple_of(half * CH, _L)
            pltpu.async_copy(src_hbm.at[pl.ds(off, CH)],
                             vsrc.at[pl.ds(boff, CH)], sem)
            eoff = pl.multiple_of(col * E + off, _L)
            pltpu.async_copy(ea_hbm.at[pl.ds(eoff, CH)],
                             vval.at[pl.ds(boff, CH)], sem)

        issue(0, 0)

        def chunk(ci, carry):
            half = lax.rem(ci, 2)
            off = pl.multiple_of(lo + ci * CH, _L)
            boff = pl.multiple_of(half * CH, _L)
            eoff = pl.multiple_of(col * E + off, _L)
            pltpu.make_async_copy(src_hbm.at[pl.ds(off, CH)],
                                  vsrc.at[pl.ds(boff, CH)], sem).wait()
            pltpu.make_async_copy(ea_hbm.at[pl.ds(eoff, CH)],
                                  vval.at[pl.ds(boff, CH)], sem).wait()

            @pl.when(ci + 1 < NCH)
            def _():
                issue(ci + 1, 1 - half)

            def grp(k, c2):
                base = pl.multiple_of(boff + k * _L * U, _L)
                for u in range(U):
                    go = pl.multiple_of(base + u * _L, _L)
                    idx = vsrc[pl.ds(go, _L)]
                    val = vval[pl.ds(go, _L)]
                    plsc.addupdate_scatter(acc, [idx], val)
                return c2
            lax.fori_loop(0, GRP_IT, grp, 0, unroll=False)
            return carry
        lax.fori_loop(0, NCH, chunk, 0, unroll=False)
        pltpu.sync_copy(acc, shared.at[pl.ds(pl.multiple_of(s * N, _L), N)])

    plsc.subcore_barrier()

    @pl.when(active)
    def _():
        SEG = N // 5
        PR = 2000
        seg = pl.multiple_of(slot * SEG, _L)
        row0 = s - slot
        _zero_vmem(segbuf, SEG)
        for part in range(5):
            poff = pl.multiple_of((row0 + part) * N + seg, _L)
            for p in range(SEG // PR):
                pltpu.sync_copy(
                    shared.at[pl.ds(pl.multiple_of(poff + p * PR, _L), PR)],
                    vred)

                def addgrp(k, c2, _p=p):
                    go = pl.multiple_of(k * _L, _L)
                    so = pl.multiple_of(_p * PR + go, _L)
                    segbuf[pl.ds(so, _L)] = segbuf[pl.ds(so, _L)] + \
                        vred[pl.ds(go, _L)]
                    return c2
                lax.fori_loop(0, PR // _L, addgrp, 0, unroll=False)
        pltpu.sync_copy(segbuf,
                        a_hbm.at[pl.ds(pl.multiple_of(col * N + seg, _L),
                                       SEG)])


def _build_b_body(E, N, CH, U, ea_hbm, sd_hbm, a_hbm, b_hbm,
                  acc, acol, vsd, vval, sem):
    wid = lax.axis_index("s") * _NC + lax.axis_index("c")
    NCH = E // CH
    GRP_IT = CH // (_L * U)

    @pl.when(wid < 25)
    def _():
        a = wid // 5
        b = wid % 5
        pltpu.sync_copy(a_hbm.at[pl.ds(pl.multiple_of(a * N, _L), N)], acol)
        _zero_vmem(acc, N)

        def issue(ci, half):
            off = pl.multiple_of(ci * CH, _L)
            boff = pl.multiple_of(half * CH, _L)
            pltpu.async_copy(sd_hbm.at[pl.ds(off, CH)],
                             vsd.at[pl.ds(boff, CH)], sem)
            eoff = pl.multiple_of(b * E + off, _L)
            pltpu.async_copy(ea_hbm.at[pl.ds(eoff, CH)],
                             vval.at[pl.ds(boff, CH)], sem)

        issue(0, 0)

        def chunk(ci, carry):
            half = lax.rem(ci, 2)
            off = pl.multiple_of(ci * CH, _L)
            boff = pl.multiple_of(half * CH, _L)
            eoff = pl.multiple_of(b * E + off, _L)
            pltpu.make_async_copy(sd_hbm.at[pl.ds(off, CH)],
                                  vsd.at[pl.ds(boff, CH)], sem).wait()
            pltpu.make_async_copy(ea_hbm.at[pl.ds(eoff, CH)],
                                  vval.at[pl.ds(boff, CH)], sem).wait()

            @pl.when(ci + 1 < NCH)
            def _():
                issue(ci + 1, 1 - half)

            def grp(k, c2):
                base = pl.multiple_of(boff + k * _L * U, _L)
                for u in range(U):
                    go = pl.multiple_of(base + u * _L, _L)
                    sd16 = vsd[pl.ds(go, _L)]
                    s16 = plsc.bitcast(
                        jnp.bitwise_and(sd16, jnp.uint32(0xFFFF)), jnp.int32)
                    d16 = plsc.bitcast(
                        jnp.right_shift(sd16, jnp.uint32(16)), jnp.int32)
                    e16 = vval[pl.ds(go, _L)]
                    g16 = plsc.load_gather(acol, [d16])
                    plsc.addupdate_scatter(acc, [s16], e16 * g16)
                return c2
            lax.fori_loop(0, GRP_IT, grp, 0, unroll=False)
            return carry
        lax.fori_loop(0, NCH, chunk, 0, unroll=False)
        pltpu.sync_copy(acc, b_hbm.at[pl.ds(pl.multiple_of(wid * N, _L), N)])


def _dense_body(n_nodes, x_ref, acl_ref, bcl_ref, w1t_ref, wr1_ref, b1_ref,
                w2t_ref, wr2_ref, b2_ref, W1_ref, bw1_ref, W2_ref, bw2_ref,
                W3_ref, bw3_ref, out_ref):
    f32 = jnp.float32
    x = x_ref[...]
    Acl = acl_ref[...]
    Bcl = bcl_ref[...]
    S1 = jnp.dot(Acl, x, preferred_element_type=f32)
    T = jnp.dot(Bcl, x, preferred_element_type=f32)
    xs = jnp.sum(x, axis=0, keepdims=True)
    As = jnp.sum(Acl, axis=1, keepdims=True)

    W1t = w1t_ref[...]
    W2t = w2t_ref[...]
    Wr1 = wr1_ref[...]
    Wr2 = wr2_ref[...]
    b1 = b1_ref[...]
    b2 = b2_ref[...]

    sm1 = jnp.zeros((1, 10), f32)
    for b in range(5):
        sm1 = sm1 + jnp.dot(S1[b:b + 1, :], W1t[b * 18:(b + 1) * 18, :],
                            preferred_element_type=f32)
    mrows = []
    for a in range(5):
        row = jnp.zeros((1, 10), f32)
        for b in range(5):
            r = a * 5 + b
            row = row + jnp.dot(T[r:r + 1, :], W1t[b * 18:(b + 1) * 18, :],
                                preferred_element_type=f32)
        mrows.append(row)
    MT = jnp.concatenate(mrows, axis=0)
    S2T = (MT + jnp.dot(S1, Wr1, preferred_element_type=f32)
           + jnp.dot(As, b1, preferred_element_type=f32))
    sm2 = jnp.zeros((1, 7), f32)
    for b in range(5):
        sm2 = sm2 + jnp.dot(S2T[b:b + 1, :], W2t[b * 10:(b + 1) * 10, :],
                            preferred_element_type=f32)
    inv_n = f32(1.0 / n_nodes)
    mean_h1 = sm1 * inv_n + jnp.dot(xs * inv_n, Wr1,
                                    preferred_element_type=f32) + b1
    pooled = sm2 * inv_n + jnp.dot(mean_h1, Wr2,
                                   preferred_element_type=f32) + b2
    z = jax.nn.relu(jnp.dot(pooled, W1_ref[...],
                            preferred_element_type=f32) + bw1_ref[...])
    z = jax.nn.relu(jnp.dot(z, W2_ref[...],
                            preferred_element_type=f32) + bw2_ref[...])
    z = jnp.dot(z, W3_ref[...], preferred_element_type=f32) + bw3_ref[...]
    out_ref[...] = jax.nn.sigmoid(z)


def kernel(x, edge_index, edge_attr, We1, be1, Wr1, b1, We2, be2, Wr2, b2,
           W1, bw1, W2, bw2, W3, bw3):
    N = x.shape[0]
    E = edge_attr.shape[0]
    src = edge_index[0]
    dst = edge_index[1]
    eaT5 = jnp.concatenate(
        [edge_attr.T, jnp.ones((1, E), jnp.float32)], axis=0).reshape(5 * E)

    sd = jnp.left_shift(dst.astype(jnp.uint32), jnp.uint32(16)) | \
        src.astype(jnp.uint32)

    CH1 = 4000
    U1 = 5
    CH2 = 6400
    U2 = 8

    sc_params = pltpu.CompilerParams(needs_layout_passes=False)

    a_cols = pl.kernel(
        functools.partial(_build_a_body, E, N, CH1, U1),
        out_type=jax.ShapeDtypeStruct((5 * N,), jnp.float32),
        mesh=_mesh(),
        compiler_params=sc_params,
        scratch_types=[
            pltpu.VMEM((N,), jnp.float32),
            pltpu.VMEM((N // 5,), jnp.float32),
            pltpu.VMEM((2 * CH1,), jnp.int32),
            pltpu.VMEM((2 * CH1,), jnp.float32),
            pltpu.VMEM((2000,), jnp.float32),
            pltpu.VMEM_SHARED((16 * N,), jnp.float32),
            pltpu.SemaphoreType.DMA,
        ],
    )(eaT5, src)

    b_cols = pl.kernel(
        functools.partial(_build_b_body, E, N, CH2, U2),
        out_type=jax.ShapeDtypeStruct((25 * N,), jnp.float32),
        mesh=_mesh(),
        compiler_params=sc_params,
        scratch_types=[
            pltpu.VMEM((N,), jnp.float32),
            pltpu.VMEM((N,), jnp.float32),
            pltpu.VMEM((2 * CH2,), jnp.uint32),
            pltpu.VMEM((2 * CH2,), jnp.float32),
            pltpu.SemaphoreType.DMA,
        ],
    )(eaT5, sd, a_cols)

    w1t = jnp.concatenate([We1, be1[None, :]], axis=0).reshape(90, 10)
    w2t = jnp.concatenate([We2, be2[None, :]], axis=0).reshape(50, 7)

    out = pl.pallas_call(
        functools.partial(_dense_body, N),
        out_shape=jax.ShapeDtypeStruct((1, 1), jnp.float32),
    )(x, a_cols.reshape(5, N), b_cols.reshape(25, N), w1t, Wr1, b1[None, :],
      w2t, Wr2, b2[None, :],
      W1, bw1[None, :], W2, bw2[None, :], W3, bw3[None, :])
    return out


# speedup vs baseline: 1.9158x; 1.9158x over previous
import functools

import jax
import jax.numpy as jnp
from jax import lax
from jax.experimental import pallas as pl
from jax.experimental.pallas import tpu as pltpu
from jax.experimental.pallas import tpu_sc as plsc

_NC = 2
_NS = 16
_L = 16


def _mesh():
    return plsc.VectorSubcoreMesh(
        core_axis_name="c", subcore_axis_name="s",
        num_cores=_NC, num_subcores=_NS)


def _zero_vmem(ref, n):
    def body(i, carry):
        off = pl.multiple_of(i * _L, _L)
        ref[pl.ds(off, _L)] = jnp.zeros((_L,), jnp.float32)
        return carry
    lax.fori_loop(0, n // _L, body, 0)


def _build_a_body(E, N, CH, U, ea_hbm, src_hbm, a_hbm,
                  acc, segbuf, vsrc, vval, vred, shared, sem):
    c = lax.axis_index("c")
    s = lax.axis_index("s")
    col_local = s // 5
    slot = s % 5
    col = jnp.where(c == 0, col_local, 3 + col_local)
    active = jnp.where(c == 0, s < 15, s < 10)
    SLOT_E = E // 5
    NCH = SLOT_E // CH
    GRP_IT = CH // (_L * U)
    lo = slot * SLOT_E

    @pl.when(active)
    def _():
        _zero_vmem(acc, N)

        def issue(ci, half):
            off = pl.multiple_of(lo + ci * CH, _L)
            boff = pl.multiple_of(half * CH, _L)
            pltpu.async_copy(src_hbm.at[pl.ds(off, CH)],
                             vsrc.at[pl.ds(boff, CH)], sem)
            eoff = pl.multiple_of(col * E + off, _L)
            pltpu.async_copy(ea_hbm.at[pl.ds(eoff, CH)],
                             vval.at[pl.ds(boff, CH)], sem)

        issue(0, 0)

        def chunk(ci, carry):
            half = lax.rem(ci, 2)
            off = pl.multiple_of(lo + ci * CH, _L)
            boff = pl.multiple_of(half * CH, _L)
            eoff = pl.multiple_of(col * E + off, _L)
            pltpu.make_async_copy(src_hbm.at[pl.ds(off, CH)],
                                  vsrc.at[pl.ds(boff, CH)], sem).wait()
            pltpu.make_async_copy(ea_hbm.at[pl.ds(eoff, CH)],
                                  vval.at[pl.ds(boff, CH)], sem).wait()

            @pl.when(ci + 1 < NCH)
            def _():
                issue(ci + 1, 1 - half)

            @plsc.parallel_loop(0, CH, _L, unroll=U)
            def grp(i):
                go = pl.multiple_of(boff + i, _L)
                idx = vsrc[pl.ds(go, _L)]
                val = vval[pl.ds(go, _L)]
                plsc.addupdate_scatter(acc, [idx], val)
            return carry
        lax.fori_loop(0, NCH, chunk, 0, unroll=False)
        pltpu.sync_copy(acc, shared.at[pl.ds(pl.multiple_of(s * N, _L), N)])

    plsc.subcore_barrier()

    @pl.when(active)
    def _():
        SEG = N // 5
        PR = 2000
        seg = pl.multiple_of(slot * SEG, _L)
        row0 = s - slot
        _zero_vmem(segbuf, SEG)
        for part in range(5):
            poff = pl.multiple_of((row0 + part) * N + seg, _L)
            for p in range(SEG // PR):
                pltpu.sync_copy(
                    shared.at[pl.ds(pl.multiple_of(poff + p * PR, _L), PR)],
                    vred)

                def addgrp(k, c2, _p=p):
                    go = pl.multiple_of(k * _L, _L)
                    so = pl.multiple_of(_p * PR + go, _L)
                    segbuf[pl.ds(so, _L)] = segbuf[pl.ds(so, _L)] + \
                        vred[pl.ds(go, _L)]
                    return c2
                lax.fori_loop(0, PR // _L, addgrp, 0, unroll=False)
        pltpu.sync_copy(segbuf,
                        a_hbm.at[pl.ds(pl.multiple_of(col * N + seg, _L),
                                       SEG)])


def _build_b_body(E, N, CH, U, ea_hbm, src_hbm, dst_hbm, a_hbm, b_hbm,
                  acc, acol, vsrc, vdst, vval, sem):
    wid = lax.axis_index("s") * _NC + lax.axis_index("c")
    NCH = E // CH

    @pl.when(wid < 25)
    def _():
        a = wid // 5
        b = wid % 5
        pltpu.sync_copy(a_hbm.at[pl.ds(pl.multiple_of(a * N, _L), N)], acol)
        _zero_vmem(acc, N)

        def issue(ci, half):
            off = pl.multiple_of(ci * CH, _L)
            boff = pl.multiple_of(half * CH, _L)
            pltpu.async_copy(src_hbm.at[pl.ds(off, CH)],
                             vsrc.at[pl.ds(boff, CH)], sem)
            pltpu.async_copy(dst_hbm.at[pl.ds(off, CH)],
                             vdst.at[pl.ds(boff, CH)], sem)
            eoff = pl.multiple_of(b * E + off, _L)
            pltpu.async_copy(ea_hbm.at[pl.ds(eoff, CH)],
                             vval.at[pl.ds(boff, CH)], sem)

        issue(0, 0)

        def chunk(ci, carry):
            half = lax.rem(ci, 2)
            off = pl.multiple_of(ci * CH, _L)
            boff = pl.multiple_of(half * CH, _L)
            eoff = pl.multiple_of(b * E + off, _L)
            pltpu.make_async_copy(src_hbm.at[pl.ds(off, CH)],
                                  vsrc.at[pl.ds(boff, CH)], sem).wait()
            pltpu.make_async_copy(dst_hbm.at[pl.ds(off, CH)],
                                  vdst.at[pl.ds(boff, CH)], sem).wait()
            pltpu.make_async_copy(ea_hbm.at[pl.ds(eoff, CH)],
                                  vval.at[pl.ds(boff, CH)], sem).wait()

            @pl.when(ci + 1 < NCH)
            def _():
                issue(ci + 1, 1 - half)

            @plsc.parallel_loop(0, CH, _L, unroll=U)
            def grp(i):
                go = pl.multiple_of(boff + i, _L)
                s16 = vsrc[pl.ds(go, _L)]
                d16 = vdst[pl.ds(go, _L)]
                e16 = vval[pl.ds(go, _L)]
                g16 = plsc.load_gather(acol, [d16])
                plsc.addupdate_scatter(acc, [s16], e16 * g16)
            return carry
        lax.fori_loop(0, NCH, chunk, 0, unroll=False)
        pltpu.sync_copy(acc, b_hbm.at[pl.ds(pl.multiple_of(wid * N, _L), N)])


def _dense_body(n_nodes, x_ref, acl_ref, bcl_ref, w1t_ref, wr1_ref, b1_ref,
                w2t_ref, wr2_ref, b2_ref, W1_ref, bw1_ref, W2_ref, bw2_ref,
                W3_ref, bw3_ref, out_ref):
    f32 = jnp.float32
    x = x_ref[...]
    Acl = acl_ref[...]
    Bcl = bcl_ref[...]
    S1 = jnp.dot(Acl, x, preferred_element_type=f32)
    T = jnp.dot(Bcl, x, preferred_element_type=f32)
    xs = jnp.sum(x, axis=0, keepdims=True)
    As = jnp.sum(Acl, axis=1, keepdims=True)

    W1t = w1t_ref[...]
    W2t = w2t_ref[...]
    Wr1 = wr1_ref[...]
    Wr2 = wr2_ref[...]
    b1 = b1_ref[...]
    b2 = b2_ref[...]

    sm1 = jnp.zeros((1, 10), f32)
    for b in range(5):
        sm1 = sm1 + jnp.dot(S1[b:b + 1, :], W1t[b * 18:(b + 1) * 18, :],
                            preferred_element_type=f32)
    mrows = []
    for a in range(5):
        row = jnp.zeros((1, 10), f32)
        for b in range(5):
            r = a * 5 + b
            row = row + jnp.dot(T[r:r + 1, :], W1t[b * 18:(b + 1) * 18, :],
                                preferred_element_type=f32)
        mrows.append(row)
    MT = jnp.concatenate(mrows, axis=0)
    S2T = (MT + jnp.dot(S1, Wr1, preferred_element_type=f32)
           + jnp.dot(As, b1, preferred_element_type=f32))
    sm2 = jnp.zeros((1, 7), f32)
    for b in range(5):
        sm2 = sm2 + jnp.dot(S2T[b:b + 1, :], W2t[b * 10:(b + 1) * 10, :],
                            preferred_element_type=f32)
    inv_n = f32(1.0 / n_nodes)
    mean_h1 = sm1 * inv_n + jnp.dot(xs * inv_n, Wr1,
                                    preferred_element_type=f32) + b1
    pooled = sm2 * inv_n + jnp.dot(mean_h1, Wr2,
                                   preferred_element_type=f32) + b2
    z = jax.nn.relu(jnp.dot(pooled, W1_ref[...],
                            preferred_element_type=f32) + bw1_ref[...])
    z = jax.nn.relu(jnp.dot(z, W2_ref[...],
                            preferred_element_type=f32) + bw2_ref[...])
    z = jnp.dot(z, W3_ref[...], preferred_element_type=f32) + bw3_ref[...]
    out_ref[...] = jax.nn.sigmoid(z)


def kernel(x, edge_index, edge_attr, We1, be1, Wr1, b1, We2, be2, Wr2, b2,
           W1, bw1, W2, bw2, W3, bw3):
    N = x.shape[0]
    E = edge_attr.shape[0]
    src = edge_index[0]
    dst = edge_index[1]
    eaT5 = jnp.concatenate(
        [edge_attr.T, jnp.ones((1, E), jnp.float32)], axis=0).reshape(5 * E)

    CH1 = 4000
    U1 = 5
    CH2 = 4000
    U2 = 8

    sc_params = pltpu.CompilerParams(needs_layout_passes=False)

    a_cols = pl.kernel(
        functools.partial(_build_a_body, E, N, CH1, U1),
        out_type=jax.ShapeDtypeStruct((5 * N,), jnp.float32),
        mesh=_mesh(),
        compiler_params=sc_params,
        scratch_types=[
            pltpu.VMEM((N,), jnp.float32),
            pltpu.VMEM((N // 5,), jnp.float32),
            pltpu.VMEM((2 * CH1,), jnp.int32),
            pltpu.VMEM((2 * CH1,), jnp.float32),
            pltpu.VMEM((2000,), jnp.float32),
            pltpu.VMEM_SHARED((16 * N,), jnp.float32),
            pltpu.SemaphoreType.DMA,
        ],
    )(eaT5, src)

    b_cols = pl.kernel(
        functools.partial(_build_b_body, E, N, CH2, U2),
        out_type=jax.ShapeDtypeStruct((25 * N,), jnp.float32),
        mesh=_mesh(),
        compiler_params=sc_params,
        scratch_types=[
            pltpu.VMEM((N,), jnp.float32),
            pltpu.VMEM((N,), jnp.float32),
            pltpu.VMEM((2 * CH2,), jnp.int32),
            pltpu.VMEM((2 * CH2,), jnp.int32),
            pltpu.VMEM((2 * CH2,), jnp.float32),
            pltpu.SemaphoreType.DMA,
        ],
    )(eaT5, src, dst, a_cols)

    w1t = jnp.concatenate([We1, be1[None, :]], axis=0).reshape(90, 10)
    w2t = jnp.concatenate([We2, be2[None, :]], axis=0).reshape(50, 7)

    out = pl.pallas_call(
        functools.partial(_dense_body, N),
        out_shape=jax.ShapeDtypeStruct((1, 1), jnp.float32),
    )(x, a_cols.reshape(5, N), b_cols.reshape(25, N), w1t, Wr1, b1[None, :],
      w2t, Wr2, b2[None, :],
      W1, bw1[None, :], W2, bw2[None, :], W3, bw3[None, :])
    return out


# speedup vs baseline: 1.9214x; 1.0029x over previous
import functools

import jax
import jax.numpy as jnp
from jax import lax
from jax.experimental import pallas as pl
from jax.experimental.pallas import tpu as pltpu
from jax.experimental.pallas import tpu_sc as plsc

_NC = 2
_NS = 16
_L = 16


def _mesh():
    return plsc.VectorSubcoreMesh(
        core_axis_name="c", subcore_axis_name="s",
        num_cores=_NC, num_subcores=_NS)


def _zero_vmem(ref, n):
    def body(i, carry):
        off = pl.multiple_of(i * _L, _L)
        ref[pl.ds(off, _L)] = jnp.zeros((_L,), jnp.float32)
        return carry
    lax.fori_loop(0, n // _L, body, 0)


def _build_a_body(E, N, CH, U, ea_hbm, src_hbm, a_hbm,
                  acc, segbuf, vsrc, vval, vred, shared, sem):
    c = lax.axis_index("c")
    s = lax.axis_index("s")
    col_local = s // 5
    slot = s % 5
    col = jnp.where(c == 0, col_local, 3 + col_local)
    active = jnp.where(c == 0, s < 15, s < 10)
    SLOT_E = E // 5
    NCH = SLOT_E // CH
    GRP_IT = CH // (_L * U)
    lo = slot * SLOT_E

    @pl.when(active)
    def _():
        _zero_vmem(acc, N)

        def issue(ci, half):
            off = pl.multiple_of(lo + ci * CH, _L)
            boff = pl.multiple_of(half * CH, _L)
            pltpu.async_copy(src_hbm.at[pl.ds(off, CH)],
                             vsrc.at[pl.ds(boff, CH)], sem)
            eoff = pl.multiple_of(col * E + off, _L)
            pltpu.async_copy(ea_hbm.at[pl.ds(eoff, CH)],
                             vval.at[pl.ds(boff, CH)], sem)

        issue(0, 0)

        def chunk(ci, carry):
            half = lax.rem(ci, 2)
            off = pl.multiple_of(lo + ci * CH, _L)
            boff = pl.multiple_of(half * CH, _L)
            eoff = pl.multiple_of(col * E + off, _L)
            pltpu.make_async_copy(src_hbm.at[pl.ds(off, CH)],
                                  vsrc.at[pl.ds(boff, CH)], sem).wait()
            pltpu.make_async_copy(ea_hbm.at[pl.ds(eoff, CH)],
                                  vval.at[pl.ds(boff, CH)], sem).wait()

            @pl.when(ci + 1 < NCH)
            def _():
                issue(ci + 1, 1 - half)

            @plsc.parallel_loop(0, CH, _L, unroll=U)
            def grp(i):
                go = pl.multiple_of(boff + i, _L)
                idx = vsrc[pl.ds(go, _L)]
                val = vval[pl.ds(go, _L)]
                plsc.addupdate_scatter(acc, [idx], val)
            return carry
        lax.fori_loop(0, NCH, chunk, 0, unroll=False)
        pltpu.sync_copy(acc, shared.at[pl.ds(pl.multiple_of(s * N, _L), N)])

    plsc.subcore_barrier()

    @pl.when(active)
    def _():
        SEG = N // 5
        PR = 2000
        seg = pl.multiple_of(slot * SEG, _L)
        row0 = s - slot
        _zero_vmem(segbuf, SEG)
        for part in range(5):
            poff = pl.multiple_of((row0 + part) * N + seg, _L)
            for p in range(SEG // PR):
                pltpu.sync_copy(
                    shared.at[pl.ds(pl.multiple_of(poff + p * PR, _L), PR)],
                    vred)

                def addgrp(k, c2, _p=p):
                    go = pl.multiple_of(k * _L, _L)
                    so = pl.multiple_of(_p * PR + go, _L)
                    segbuf[pl.ds(so, _L)] = segbuf[pl.ds(so, _L)] + \
                        vred[pl.ds(go, _L)]
                    return c2
                lax.fori_loop(0, PR // _L, addgrp, 0, unroll=False)
        pltpu.sync_copy(segbuf,
                        a_hbm.at[pl.ds(pl.multiple_of(col * N + seg, _L),
                                       SEG)])


def _build_b_body(E, N, CH, U, ea_hbm, src_hbm, dst_hbm, a_hbm, b_hbm,
                  acc, acol, vsrc, vdst, vval, sem):
    wid = lax.axis_index("s") * _NC + lax.axis_index("c")
    NCH = E // CH

    @pl.when(wid < 25)
    def _():
        a = wid // 5
        b = wid % 5

        def issue(ci, half):
            off = pl.multiple_of(ci * CH, _L)
            boff = pl.multiple_of(half * CH, _L)
            pltpu.async_copy(src_hbm.at[pl.ds(off, CH)],
                             vsrc.at[pl.ds(boff, CH)], sem)
            pltpu.async_copy(dst_hbm.at[pl.ds(off, CH)],
                             vdst.at[pl.ds(boff, CH)], sem)
            eoff = pl.multiple_of(b * E + off, _L)
            pltpu.async_copy(ea_hbm.at[pl.ds(eoff, CH)],
                             vval.at[pl.ds(boff, CH)], sem)

        issue(0, 0)
        pltpu.sync_copy(a_hbm.at[pl.ds(pl.multiple_of(a * N, _L), N)], acol)
        _zero_vmem(acc, N)

        def chunk(ci, carry):
            half = lax.rem(ci, 2)
            off = pl.multiple_of(ci * CH, _L)
            boff = pl.multiple_of(half * CH, _L)
            eoff = pl.multiple_of(b * E + off, _L)
            pltpu.make_async_copy(src_hbm.at[pl.ds(off, CH)],
                                  vsrc.at[pl.ds(boff, CH)], sem).wait()
            pltpu.make_async_copy(dst_hbm.at[pl.ds(off, CH)],
                                  vdst.at[pl.ds(boff, CH)], sem).wait()
            pltpu.make_async_copy(ea_hbm.at[pl.ds(eoff, CH)],
                                  vval.at[pl.ds(boff, CH)], sem).wait()

            @pl.when(ci + 1 < NCH)
            def _():
                issue(ci + 1, 1 - half)

            @plsc.parallel_loop(0, CH, _L, unroll=U)
            def grp(i):
                go = pl.multiple_of(boff + i, _L)
                s16 = vsrc[pl.ds(go, _L)]
                d16 = vdst[pl.ds(go, _L)]
                e16 = vval[pl.ds(go, _L)]
                g16 = plsc.load_gather(acol, [d16])
                plsc.addupdate_scatter(acc, [s16], e16 * g16)
            return carry
        lax.fori_loop(0, NCH, chunk, 0, unroll=False)
        pltpu.sync_copy(acc, b_hbm.at[pl.ds(pl.multiple_of(wid * N, _L), N)])


def _dense_body(n_nodes, x_ref, acl_ref, bcl_ref, w1t_ref, wr1_ref, b1_ref,
                w2t_ref, wr2_ref, b2_ref, W1_ref, bw1_ref, W2_ref, bw2_ref,
                W3_ref, bw3_ref, out_ref):
    f32 = jnp.float32
    x = x_ref[...]
    Acl = acl_ref[...]
    Bcl = bcl_ref[...]
    S1 = jnp.dot(Acl, x, preferred_element_type=f32)
    T = jnp.dot(Bcl, x, preferred_element_type=f32)
    xs = jnp.sum(x, axis=0, keepdims=True)
    As = jnp.sum(Acl, axis=1, keepdims=True)

    W1t = w1t_ref[...]
    W2t = w2t_ref[...]
    Wr1 = wr1_ref[...]
    Wr2 = wr2_ref[...]
    b1 = b1_ref[...]
    b2 = b2_ref[...]

    sm1 = jnp.zeros((1, 10), f32)
    for b in range(5):
        sm1 = sm1 + jnp.dot(S1[b:b + 1, :], W1t[b * 18:(b + 1) * 18, :],
                            preferred_element_type=f32)
    mrows = []
    for a in range(5):
        row = jnp.zeros((1, 10), f32)
        for b in range(5):
            r = a * 5 + b
            row = row + jnp.dot(T[r:r + 1, :], W1t[b * 18:(b + 1) * 18, :],
                                preferred_element_type=f32)
        mrows.append(row)
    MT = jnp.concatenate(mrows, axis=0)
    S2T = (MT + jnp.dot(S1, Wr1, preferred_element_type=f32)
           + jnp.dot(As, b1, preferred_element_type=f32))
    sm2 = jnp.zeros((1, 7), f32)
    for b in range(5):
        sm2 = sm2 + jnp.dot(S2T[b:b + 1, :], W2t[b * 10:(b + 1) * 10, :],
                            preferred_element_type=f32)
    inv_n = f32(1.0 / n_nodes)
    mean_h1 = sm1 * inv_n + jnp.dot(xs * inv_n, Wr1,
                                    preferred_element_type=f32) + b1
    pooled = sm2 * inv_n + jnp.dot(mean_h1, Wr2,
                                   preferred_element_type=f32) + b2
    z = jax.nn.relu(jnp.dot(pooled, W1_ref[...],
                            preferred_element_type=f32) + bw1_ref[...])
    z = jax.nn.relu(jnp.dot(z, W2_ref[...],
                            preferred_element_type=f32) + bw2_ref[...])
    z = jnp.dot(z, W3_ref[...], preferred_element_type=f32) + bw3_ref[...]
    out_ref[...] = jax.nn.sigmoid(z)


def kernel(x, edge_index, edge_attr, We1, be1, Wr1, b1, We2, be2, Wr2, b2,
           W1, bw1, W2, bw2, W3, bw3):
    N = x.shape[0]
    E = edge_attr.shape[0]
    src = edge_index[0]
    dst = edge_index[1]
    eaT5 = jnp.concatenate(
        [edge_attr.T, jnp.ones((1, E), jnp.float32)], axis=0).reshape(5 * E)

    CH1 = 4000
    U1 = 5
    CH2 = 4000
    U2 = 8

    sc_params = pltpu.CompilerParams(needs_layout_passes=False)

    a_cols = pl.kernel(
        functools.partial(_build_a_body, E, N, CH1, U1),
        out_type=jax.ShapeDtypeStruct((5 * N,), jnp.float32),
        mesh=_mesh(),
        compiler_params=sc_params,
        scratch_types=[
            pltpu.VMEM((N,), jnp.float32),
            pltpu.VMEM((N // 5,), jnp.float32),
            pltpu.VMEM((2 * CH1,), jnp.int32),
            pltpu.VMEM((2 * CH1,), jnp.float32),
            pltpu.VMEM((2000,), jnp.float32),
            pltpu.VMEM_SHARED((16 * N,), jnp.float32),
            pltpu.SemaphoreType.DMA,
        ],
    )(eaT5, src)

    b_cols = pl.kernel(
        functools.partial(_build_b_body, E, N, CH2, U2),
        out_type=jax.ShapeDtypeStruct((25 * N,), jnp.float32),
        mesh=_mesh(),
        compiler_params=sc_params,
        scratch_types=[
            pltpu.VMEM((N,), jnp.float32),
            pltpu.VMEM((N,), jnp.float32),
            pltpu.VMEM((2 * CH2,), jnp.int32),
            pltpu.VMEM((2 * CH2,), jnp.int32),
            pltpu.VMEM((2 * CH2,), jnp.float32),
            pltpu.SemaphoreType.DMA,
        ],
    )(eaT5, src, dst, a_cols)

    w1t = jnp.concatenate([We1, be1[None, :]], axis=0).reshape(90, 10)
    w2t = jnp.concatenate([We2, be2[None, :]], axis=0).reshape(50, 7)

    out = pl.pallas_call(
        functools.partial(_dense_body, N),
        out_shape=jax.ShapeDtypeStruct((1, 1), jnp.float32),
    )(x, a_cols.reshape(5, N), b_cols.reshape(25, N), w1t, Wr1, b1[None, :],
      w2t, Wr2, b2[None, :],
      W1, bw1[None, :], W2, bw2[None, :], W3, bw3[None, :])
    return out
